# Initial kernel scaffold; baseline (speedup 1.0000x reference)
import jax
import jax.numpy as jnp
from jax.experimental import pallas as pl


def kernel(x, table):
    raise NotImplementedError("write your pallas kernel here")


# speedup vs baseline: 1.4351x; 1.4351x over previous
import functools

import jax
import jax.numpy as jnp
from jax import lax
from jax.experimental import pallas as pl
from jax.experimental.pallas import tpu as pltpu
from jax.experimental.pallas import tpu_sc as plsc

NC, NS = 2, 16
NW = NC * NS
BATCH, SEQ, D = 4, 2048, 512
B = BATCH * SEQ
BPW = B // NW
CHUNK = 64
NCHUNK = BPW // CHUNK
NBUF = 3

_mesh = plsc.VectorSubcoreMesh(core_axis_name="c", subcore_axis_name="s")


@functools.partial(
    pl.kernel,
    mesh=_mesh,
    out_type=jax.ShapeDtypeStruct((B, D), jnp.float32),
    scratch_types=[
        pltpu.VMEM((BPW,), jnp.int32),
        *[pltpu.VMEM((CHUNK, D), jnp.float32) for _ in range(NBUF)],
        *[pltpu.SemaphoreType.DMA for _ in range(2 * NBUF)],
    ],
)
def _embed_gather(idx_hbm, table_hbm, out_hbm, idx_v, *scratch):
    bufs = scratch[:NBUF]
    gsems = scratch[NBUF:2 * NBUF]
    osems = scratch[2 * NBUF:]

    wid = lax.axis_index("s") * NC + lax.axis_index("c")
    base = wid * BPW
    pltpu.sync_copy(idx_hbm.at[pl.ds(base, BPW)], idx_v)

    def start_gather(c):
        b = c % NBUF
        return pltpu.async_copy(
            table_hbm.at[idx_v.at[pl.ds(c * CHUNK, CHUNK)]], bufs[b], gsems[b])

    gh = [None] * NCHUNK
    oh = [None] * NCHUNK
    for c in range(min(NBUF, NCHUNK)):
        gh[c] = start_gather(c)
    for c in range(NCHUNK):
        b = c % NBUF
        gh[c].wait()
        oh[c] = pltpu.async_copy(
            bufs[b], out_hbm.at[pl.ds(base + c * CHUNK, CHUNK)], osems[b])
        nxt = c + NBUF
        if nxt < NCHUNK:
            oh[c].wait()
            oh[c] = None
            gh[nxt] = start_gather(nxt)
    for c in range(NCHUNK):
        if oh[c] is not None:
            oh[c].wait()


def kernel(x, table):
    idx = x.reshape(-1).astype(jnp.int32)
    out = _embed_gather(idx, table)
    return out.reshape(*x.shape, D)
